# baseline (speedup 1.0000x reference)
import functools

import jax
import jax.numpy as jnp
from jax import lax
from jax.experimental import pallas as pl
from jax.experimental.pallas import tpu as pltpu
from jax.experimental.pallas import tpu_sc as plsc

N = 10000
E = 160000
D = 256
HD = 128
NC, NS = 2, 16
EPS = 1e-5

EC = 40
NCHUNK = E // NS // EC
WIN = 25
NWIN = NCHUNK // WIN
DC = 40
NDCHUNK = E // NC // NS // DC

_BN = 1000
_GRID = N // _BN

_mesh = plsc.VectorSubcoreMesh(core_axis_name="c", subcore_axis_name="s",
                               num_cores=NC, num_subcores=NS)


@functools.partial(
    pl.kernel,
    out_type=jax.ShapeDtypeStruct((NC, N, HD), jnp.float32),
    mesh=_mesh,
    scratch_types=[
        pltpu.VMEM((NDCHUNK, DC), jnp.int32),
        pltpu.VMEM((DC, HD), jnp.float32),
        pltpu.VMEM_SHARED((N, HD), jnp.float32),
        pltpu.SemaphoreType.DMA,
    ],
)
def _deg_kernel(dst_hbm, ones_hbm, zeros_hbm, out_hbm, idx_v, ones_v, deg_sp,
                dsem):
    c = lax.axis_index("c")
    s = lax.axis_index("s")

    @pl.when(s < 10)
    def _zero():
        pltpu.sync_copy(zeros_hbm, deg_sp.at[pl.ds(s * 1000, 1000)])

    pltpu.sync_copy(ones_hbm, ones_v)
    pltpu.sync_copy(dst_hbm.at[c, s], idx_v)
    plsc.subcore_barrier()

    def chunk(i, carry):
        for b in range(5):
            pltpu.async_copy(ones_v, deg_sp.at[idx_v.at[5 * i + b]], dsem,
                             add=True)
        for b in range(5):
            pltpu.make_async_copy(ones_v, deg_sp.at[idx_v.at[0]],
                                  dsem).wait()
        return carry

    lax.fori_loop(0, NDCHUNK // 5, chunk, 0)
    plsc.subcore_barrier()

    @pl.when(s < 10)
    def _write():
        pltpu.sync_copy(deg_sp.at[pl.ds(s * 1000, 1000)],
                        out_hbm.at[c, pl.ds(s * 1000, 1000)])


@functools.partial(
    pl.kernel,
    out_type=jax.ShapeDtypeStruct((NC, N, HD), jnp.float32),
    mesh=_mesh,
    scratch_types=[
        pltpu.VMEM((WIN, EC), jnp.int32),
        pltpu.VMEM((WIN, EC), jnp.int32),
        pltpu.VMEM((EC, HD), jnp.float32),
        pltpu.VMEM((EC, HD), jnp.float32),
        pltpu.VMEM((EC, HD), jnp.float32),
        pltpu.VMEM((EC, HD), jnp.float32),
        pltpu.VMEM_SHARED((N, HD), jnp.float32),
        pltpu.SemaphoreType.DMA,
        pltpu.SemaphoreType.DMA,
        pltpu.SemaphoreType.DMA,
        pltpu.SemaphoreType.DMA,
        pltpu.SemaphoreType.DMA,
        pltpu.SemaphoreType.DMA,
        pltpu.SemaphoreType.DMA,
        pltpu.SemaphoreType.DMA,
    ],
)
def _edge_kernel(y_hbm, src_hbm, dst_hbm, zeros_hbm, out_hbm,
                 src_v, dst_v, rows0, rows1, rows2, rows3, acc_sp,
                 gsem0, gsem1, gsem2, gsem3, ssem0, ssem1, ssem2, ssem3):
    c = lax.axis_index("c")
    s = lax.axis_index("s")
    rows = (rows0, rows1, rows2, rows3)
    gsem = (gsem0, gsem1, gsem2, gsem3)
    ssem = (ssem0, ssem1, ssem2, ssem3)

    @pl.when(s < 10)
    def _zero():
        pltpu.sync_copy(zeros_hbm, acc_sp.at[pl.ds(s * 1000, 1000)])

    plsc.subcore_barrier()

    def start_gather(b, j):
        pltpu.async_copy(y_hbm.at[src_v.at[j]], rows[b], gsem[b])

    def wait_gather(b):
        pltpu.make_async_copy(y_hbm.at[src_v.at[0]], rows[b], gsem[b]).wait()

    def start_scatter(b, j):
        pltpu.async_copy(rows[b], acc_sp.at[dst_v.at[j]], ssem[b], add=True)

    def wait_scatter(b):
        pltpu.make_async_copy(rows[b], acc_sp.at[dst_v.at[0]],
                              ssem[b]).wait()

    def window(w, carry):
        pltpu.sync_copy(src_hbm.at[c, s, w], src_v)
        pltpu.sync_copy(dst_hbm.at[s, w], dst_v)
        start_gather(0, 0)
        start_gather(1, 1)
        start_gather(2, 2)

        def quad(i, cc):
            for b in (0, 1, 2, 3):
                k = 4 * i + b
                bp = (b + 3) % 4

                @pl.when(k >= 1)
                def _free():
                    wait_scatter(bp)

                @pl.when(k + 3 <= WIN - 1)
                def _prefetch():
                    start_gather(bp, k + 3)

                wait_gather(b)
                start_scatter(b, k)
            return cc

        lax.fori_loop(0, (WIN - 1) // 4, quad, 0)
        wait_scatter(3)
        wait_gather(0)
        start_scatter(0, WIN - 1)
        wait_scatter(0)
        return carry

    lax.fori_loop(0, NWIN, window, 0)
    plsc.subcore_barrier()

    @pl.when(s < 10)
    def _write():
        pltpu.sync_copy(acc_sp.at[pl.ds(s * 1000, 1000)],
                        out_hbm.at[c, pl.ds(s * 1000, 1000)])


def _k1_body(x_ref, w_ref, degp_ref, y_ref):
    d = degp_ref[...]
    dinv = lax.rsqrt(d[0, :, 0:1] + d[1, :, 0:1] + 1.0)
    y = jnp.dot(x_ref[...], w_ref[...],
                preferred_element_type=jnp.float32) * dinv
    y_ref[0] = y[:, :HD]
    y_ref[1] = y[:, HD:]


def _k1(x, w, degp):
    return pl.pallas_call(
        _k1_body,
        grid=(_GRID,),
        in_specs=[
            pl.BlockSpec((_BN, D), lambda i: (i, 0)),
            pl.BlockSpec((D, D), lambda i: (0, 0)),
            pl.BlockSpec((NC, _BN, HD), lambda i: (0, i, 0)),
        ],
        out_specs=pl.BlockSpec((NC, _BN, HD), lambda i: (0, i, 0)),
        out_shape=jax.ShapeDtypeStruct((NC, N, HD), jnp.float32),
    )(x, w, degp)


def _k2_body(acc_ref, y_ref, degp_ref, b_ref, z_ref, stats_ref):
    i = pl.program_id(0)
    d = degp_ref[...]
    dinv = lax.rsqrt(d[0, :, 0:1] + d[1, :, 0:1] + 1.0)
    acc = jnp.concatenate([acc_ref[0], acc_ref[1]], axis=1)
    y = jnp.concatenate([y_ref[0], y_ref[1]], axis=1)
    z = dinv * (acc + y) + b_ref[...]
    z_ref[...] = z
    st = jnp.concatenate([jnp.sum(z, 0, keepdims=True),
                          jnp.sum(z * z, 0, keepdims=True)], axis=0)

    @pl.when(i == 0)
    def _init():
        stats_ref[...] = st

    @pl.when(i > 0)
    def _acc():
        stats_ref[...] += st


def _k2(acc, y, degp, b):
    return pl.pallas_call(
        _k2_body,
        grid=(_GRID,),
        in_specs=[
            pl.BlockSpec((NC, _BN, HD), lambda i: (0, i, 0)),
            pl.BlockSpec((NC, _BN, HD), lambda i: (0, i, 0)),
            pl.BlockSpec((NC, _BN, HD), lambda i: (0, i, 0)),
            pl.BlockSpec((1, D), lambda i: (0, 0)),
        ],
        out_specs=[
            pl.BlockSpec((_BN, D), lambda i: (i, 0)),
            pl.BlockSpec((2, D), lambda i: (0, 0)),
        ],
        out_shape=[
            jax.ShapeDtypeStruct((N, D), jnp.float32),
            jax.ShapeDtypeStruct((2, D), jnp.float32),
        ],
    )(acc, y, degp, b)


def _k3_body(z_ref, stats_ref, g_ref, bt_ref, w_ref, degp_ref, y_ref):
    mu = stats_ref[0:1] / N
    var = stats_ref[1:2] / N - mu * mu
    scale = g_ref[...] * lax.rsqrt(var + EPS)
    h = jnp.maximum((z_ref[...] - mu) * scale + bt_ref[...], 0.0)
    d = degp_ref[...]
    dinv = lax.rsqrt(d[0, :, 0:1] + d[1, :, 0:1] + 1.0)
    y = jnp.dot(h, w_ref[...], preferred_element_type=jnp.float32) * dinv
    y_ref[0] = y[:, :HD]
    y_ref[1] = y[:, HD:]


def _k3(z, stats, gamma, beta, w, degp):
    return pl.pallas_call(
        _k3_body,
        grid=(_GRID,),
        in_specs=[
            pl.BlockSpec((_BN, D), lambda i: (i, 0)),
            pl.BlockSpec((2, D), lambda i: (0, 0)),
            pl.BlockSpec((1, D), lambda i: (0, 0)),
            pl.BlockSpec((1, D), lambda i: (0, 0)),
            pl.BlockSpec((D, D), lambda i: (0, 0)),
            pl.BlockSpec((NC, _BN, HD), lambda i: (0, i, 0)),
        ],
        out_specs=pl.BlockSpec((NC, _BN, HD), lambda i: (0, i, 0)),
        out_shape=jax.ShapeDtypeStruct((NC, N, HD), jnp.float32),
    )(z, stats, gamma, beta, w, degp)


def _k5_body(z_ref, stats_ref, g_ref, bt_ref, x_ref, out_ref):
    mu = stats_ref[0:1] / N
    var = stats_ref[1:2] / N - mu * mu
    scale = g_ref[...] * lax.rsqrt(var + EPS)
    out_ref[...] = jnp.maximum((z_ref[...] - mu) * scale + bt_ref[...]
                               + x_ref[...], 0.0)


def _k5(z, stats, gamma, beta, x):
    return pl.pallas_call(
        _k5_body,
        grid=(_GRID,),
        in_specs=[
            pl.BlockSpec((_BN, D), lambda i: (i, 0)),
            pl.BlockSpec((2, D), lambda i: (0, 0)),
            pl.BlockSpec((1, D), lambda i: (0, 0)),
            pl.BlockSpec((1, D), lambda i: (0, 0)),
            pl.BlockSpec((_BN, D), lambda i: (i, 0)),
        ],
        out_specs=pl.BlockSpec((_BN, D), lambda i: (i, 0)),
        out_shape=jax.ShapeDtypeStruct((N, D), jnp.float32),
    )(z, stats, gamma, beta, x)


def kernel(x, edge_index, W1, b1, gamma1, beta1, W2, b2, gamma2, beta2):
    src = edge_index[0].astype(jnp.int32)
    dst = edge_index[1].astype(jnp.int32)

    dst_deg = dst.reshape(NC, NS, NDCHUNK, DC)
    src2 = jnp.stack([src, src + N]).reshape(NC, NS, NWIN, WIN, EC)
    dst_sc = dst.reshape(NS, NWIN, WIN, EC)

    ones128 = jnp.ones((DC, HD), jnp.float32)
    zeros1k = jnp.zeros((1000, HD), jnp.float32)
    zerosrows = jnp.zeros((1000, HD), jnp.float32)

    degp3 = _deg_kernel(dst_deg, ones128, zeros1k)

    b1r = b1.reshape(1, D)
    b2r = b2.reshape(1, D)
    g1 = gamma1.reshape(1, D)
    g2 = gamma2.reshape(1, D)
    bt1 = beta1.reshape(1, D)
    bt2 = beta2.reshape(1, D)

    y1 = _k1(x, W1, degp3)
    acc1 = _edge_kernel(y1.reshape(NC * N, HD), src2, dst_sc, zerosrows)
    z1, st1 = _k2(acc1, y1, degp3, b1r)
    y2 = _k3(z1, st1, g1, bt1, W2, degp3)
    acc2 = _edge_kernel(y2.reshape(NC * N, HD), src2, dst_sc, zerosrows)
    z2, st2 = _k2(acc2, y2, degp3, b2r)
    return _k5(z2, st2, g2, bt2, x)


# speedup vs baseline: 1.1130x; 1.1130x over previous
import functools

import jax
import jax.numpy as jnp
from jax import lax
from jax.experimental import pallas as pl
from jax.experimental.pallas import tpu as pltpu
from jax.experimental.pallas import tpu_sc as plsc

N = 10000
E = 160000
D = 256
HD = 128
NC, NS = 2, 16
EPS = 1e-5

EC = 80
NCHUNK = E // NS // EC
WIN = 25
NWIN = NCHUNK // WIN
DC = 40
NDCHUNK = E // NC // NS // DC

_BN = 2000
_GRID = N // _BN

_mesh = plsc.VectorSubcoreMesh(core_axis_name="c", subcore_axis_name="s",
                               num_cores=NC, num_subcores=NS)


@functools.partial(
    pl.kernel,
    out_type=jax.ShapeDtypeStruct((NC, N, HD), jnp.float32),
    mesh=_mesh,
    scratch_types=[
        pltpu.VMEM((NDCHUNK, DC), jnp.int32),
        pltpu.VMEM((DC, HD), jnp.float32),
        pltpu.VMEM_SHARED((N, HD), jnp.float32),
        pltpu.SemaphoreType.DMA,
    ],
)
def _deg_kernel(dst_hbm, ones_hbm, zeros_hbm, out_hbm, idx_v, ones_v, deg_sp,
                dsem):
    c = lax.axis_index("c")
    s = lax.axis_index("s")

    @pl.when(s < 10)
    def _zero():
        pltpu.sync_copy(zeros_hbm, deg_sp.at[pl.ds(s * 1000, 1000)])

    pltpu.sync_copy(ones_hbm, ones_v)
    pltpu.sync_copy(dst_hbm.at[c, s], idx_v)
    plsc.subcore_barrier()

    def chunk(i, carry):
        for b in range(5):
            pltpu.async_copy(ones_v, deg_sp.at[idx_v.at[5 * i + b]], dsem,
                             add=True)
        for b in range(5):
            pltpu.make_async_copy(ones_v, deg_sp.at[idx_v.at[0]],
                                  dsem).wait()
        return carry

    lax.fori_loop(0, NDCHUNK // 5, chunk, 0)
    plsc.subcore_barrier()

    @pl.when(s < 10)
    def _write():
        pltpu.sync_copy(deg_sp.at[pl.ds(s * 1000, 1000)],
                        out_hbm.at[c, pl.ds(s * 1000, 1000)])


@functools.partial(
    pl.kernel,
    out_type=jax.ShapeDtypeStruct((NC, N, HD), jnp.float32),
    mesh=_mesh,
    scratch_types=[
        pltpu.VMEM((WIN, EC), jnp.int32),
        pltpu.VMEM((WIN, EC), jnp.int32),
        pltpu.VMEM((EC, HD), jnp.float32),
        pltpu.VMEM((EC, HD), jnp.float32),
        pltpu.VMEM((EC, HD), jnp.float32),
        pltpu.VMEM_SHARED((N, HD), jnp.float32),
        pltpu.SemaphoreType.DMA,
        pltpu.SemaphoreType.DMA,
        pltpu.SemaphoreType.DMA,
        pltpu.SemaphoreType.DMA,
        pltpu.SemaphoreType.DMA,
        pltpu.SemaphoreType.DMA,
    ],
)
def _edge_kernel(y_hbm, src_hbm, dst_hbm, zeros_hbm, out_hbm,
                 src_v, dst_v, rows0, rows1, rows2, acc_sp,
                 gsem0, gsem1, gsem2, ssem0, ssem1, ssem2):
    c = lax.axis_index("c")
    s = lax.axis_index("s")
    rows = (rows0, rows1, rows2)
    gsem = (gsem0, gsem1, gsem2)
    ssem = (ssem0, ssem1, ssem2)

    @pl.when(s < 10)
    def _zero():
        pltpu.sync_copy(zeros_hbm, acc_sp.at[pl.ds(s * 1000, 1000)])

    plsc.subcore_barrier()

    def start_gather(b, j):
        pltpu.async_copy(y_hbm.at[src_v.at[j]], rows[b], gsem[b])

    def wait_gather(b):
        pltpu.make_async_copy(y_hbm.at[src_v.at[0]], rows[b], gsem[b]).wait()

    def start_scatter(b, j):
        pltpu.async_copy(rows[b], acc_sp.at[dst_v.at[j]], ssem[b], add=True)

    def wait_scatter(b):
        pltpu.make_async_copy(rows[b], acc_sp.at[dst_v.at[0]],
                              ssem[b]).wait()

    def window(w, carry):
        pltpu.sync_copy(src_hbm.at[c, s, w], src_v)
        pltpu.sync_copy(dst_hbm.at[s, w], dst_v)
        start_gather(0, 0)
        start_gather(1, 1)

        def triple(i, cc):
            for b in (0, 1, 2):
                k = 3 * i + b
                bp = (b + 2) % 3

                @pl.when(k >= 1)
                def _free():
                    wait_scatter(bp)

                @pl.when(k + 2 <= WIN - 1)
                def _prefetch():
                    start_gather(bp, k + 2)

                wait_gather(b)
                start_scatter(b, k)
            return cc

        lax.fori_loop(0, (WIN - 1) // 3, triple, 0)
        wait_scatter(2)
        wait_gather(0)
        start_scatter(0, WIN - 1)
        wait_scatter(0)
        return carry

    lax.fori_loop(0, NWIN, window, 0)
    plsc.subcore_barrier()

    @pl.when(s < 10)
    def _write():
        pltpu.sync_copy(acc_sp.at[pl.ds(s * 1000, 1000)],
                        out_hbm.at[c, pl.ds(s * 1000, 1000)])


def _k1_body(x_ref, w_ref, degp_ref, y_ref):
    d = degp_ref[...]
    dinv = lax.rsqrt(d[0, :, 0:1] + d[1, :, 0:1] + 1.0)
    y = jnp.dot(x_ref[...], w_ref[...],
                preferred_element_type=jnp.float32) * dinv
    y_ref[0] = y[:, :HD]
    y_ref[1] = y[:, HD:]


def _k1(x, w, degp):
    return pl.pallas_call(
        _k1_body,
        grid=(_GRID,),
        in_specs=[
            pl.BlockSpec((_BN, D), lambda i: (i, 0)),
            pl.BlockSpec((D, D), lambda i: (0, 0)),
            pl.BlockSpec((NC, _BN, HD), lambda i: (0, i, 0)),
        ],
        out_specs=pl.BlockSpec((NC, _BN, HD), lambda i: (0, i, 0)),
        out_shape=jax.ShapeDtypeStruct((NC, N, HD), jnp.float32),
    )(x, w, degp)


def _k2_body(acc_ref, y_ref, degp_ref, b_ref, z_ref, stats_ref):
    i = pl.program_id(0)
    d = degp_ref[...]
    dinv = lax.rsqrt(d[0, :, 0:1] + d[1, :, 0:1] + 1.0)
    acc = jnp.concatenate([acc_ref[0], acc_ref[1]], axis=1)
    y = jnp.concatenate([y_ref[0], y_ref[1]], axis=1)
    z = dinv * (acc + y) + b_ref[...]
    z_ref[...] = z
    st = jnp.concatenate([jnp.sum(z, 0, keepdims=True),
                          jnp.sum(z * z, 0, keepdims=True)], axis=0)

    @pl.when(i == 0)
    def _init():
        stats_ref[...] = st

    @pl.when(i > 0)
    def _acc():
        stats_ref[...] += st


def _k2(acc, y, degp, b):
    return pl.pallas_call(
        _k2_body,
        grid=(_GRID,),
        in_specs=[
            pl.BlockSpec((NC, _BN, HD), lambda i: (0, i, 0)),
            pl.BlockSpec((NC, _BN, HD), lambda i: (0, i, 0)),
            pl.BlockSpec((NC, _BN, HD), lambda i: (0, i, 0)),
            pl.BlockSpec((1, D), lambda i: (0, 0)),
        ],
        out_specs=[
            pl.BlockSpec((_BN, D), lambda i: (i, 0)),
            pl.BlockSpec((2, D), lambda i: (0, 0)),
        ],
        out_shape=[
            jax.ShapeDtypeStruct((N, D), jnp.float32),
            jax.ShapeDtypeStruct((2, D), jnp.float32),
        ],
    )(acc, y, degp, b)


def _k3_body(z_ref, stats_ref, g_ref, bt_ref, w_ref, degp_ref, y_ref):
    mu = stats_ref[0:1] / N
    var = stats_ref[1:2] / N - mu * mu
    scale = g_ref[...] * lax.rsqrt(var + EPS)
    h = jnp.maximum((z_ref[...] - mu) * scale + bt_ref[...], 0.0)
    d = degp_ref[...]
    dinv = lax.rsqrt(d[0, :, 0:1] + d[1, :, 0:1] + 1.0)
    y = jnp.dot(h, w_ref[...], preferred_element_type=jnp.float32) * dinv
    y_ref[0] = y[:, :HD]
    y_ref[1] = y[:, HD:]


def _k3(z, stats, gamma, beta, w, degp):
    return pl.pallas_call(
        _k3_body,
        grid=(_GRID,),
        in_specs=[
            pl.BlockSpec((_BN, D), lambda i: (i, 0)),
            pl.BlockSpec((2, D), lambda i: (0, 0)),
            pl.BlockSpec((1, D), lambda i: (0, 0)),
            pl.BlockSpec((1, D), lambda i: (0, 0)),
            pl.BlockSpec((D, D), lambda i: (0, 0)),
            pl.BlockSpec((NC, _BN, HD), lambda i: (0, i, 0)),
        ],
        out_specs=pl.BlockSpec((NC, _BN, HD), lambda i: (0, i, 0)),
        out_shape=jax.ShapeDtypeStruct((NC, N, HD), jnp.float32),
    )(z, stats, gamma, beta, w, degp)


def _k5_body(z_ref, stats_ref, g_ref, bt_ref, x_ref, out_ref):
    mu = stats_ref[0:1] / N
    var = stats_ref[1:2] / N - mu * mu
    scale = g_ref[...] * lax.rsqrt(var + EPS)
    out_ref[...] = jnp.maximum((z_ref[...] - mu) * scale + bt_ref[...]
                               + x_ref[...], 0.0)


def _k5(z, stats, gamma, beta, x):
    return pl.pallas_call(
        _k5_body,
        grid=(_GRID,),
        in_specs=[
            pl.BlockSpec((_BN, D), lambda i: (i, 0)),
            pl.BlockSpec((2, D), lambda i: (0, 0)),
            pl.BlockSpec((1, D), lambda i: (0, 0)),
            pl.BlockSpec((1, D), lambda i: (0, 0)),
            pl.BlockSpec((_BN, D), lambda i: (i, 0)),
        ],
        out_specs=pl.BlockSpec((_BN, D), lambda i: (i, 0)),
        out_shape=jax.ShapeDtypeStruct((N, D), jnp.float32),
    )(z, stats, gamma, beta, x)


def kernel(x, edge_index, W1, b1, gamma1, beta1, W2, b2, gamma2, beta2):
    src = edge_index[0].astype(jnp.int32)
    dst = edge_index[1].astype(jnp.int32)

    dst_deg = dst.reshape(NC, NS, NDCHUNK, DC)
    src2 = jnp.stack([src, src + N]).reshape(NC, NS, NWIN, WIN, EC)
    dst_sc = dst.reshape(NS, NWIN, WIN, EC)

    ones128 = jnp.ones((DC, HD), jnp.float32)
    zeros1k = jnp.zeros((1000, HD), jnp.float32)
    zerosrows = jnp.zeros((1000, HD), jnp.float32)

    degp3 = _deg_kernel(dst_deg, ones128, zeros1k)

    b1r = b1.reshape(1, D)
    b2r = b2.reshape(1, D)
    g1 = gamma1.reshape(1, D)
    g2 = gamma2.reshape(1, D)
    bt1 = beta1.reshape(1, D)
    bt2 = beta2.reshape(1, D)

    y1 = _k1(x, W1, degp3)
    acc1 = _edge_kernel(y1.reshape(NC * N, HD), src2, dst_sc, zerosrows)
    z1, st1 = _k2(acc1, y1, degp3, b1r)
    y2 = _k3(z1, st1, g1, bt1, W2, degp3)
    acc2 = _edge_kernel(y2.reshape(NC * N, HD), src2, dst_sc, zerosrows)
    z2, st2 = _k2(acc2, y2, degp3, b2r)
    return _k5(z2, st2, g2, bt2, x)
